# baseline (speedup 1.0000x reference)
import jax
import jax.numpy as jnp
import numpy as np
from jax.experimental import pallas as pl
from jax.experimental.pallas import tpu as pltpu

N = 10000
E = 320000
D = 128
G = 64


def _transformer_conv(x, src, dst, ea, p, heads, out):
    Nn = x.shape[0]
    q = (x @ p['Wq'] + p['bq'])[dst].reshape(-1, heads, out)
    k = (x @ p['Wk'] + p['bk'])[src].reshape(-1, heads, out)
    v = (x @ p['Wv'] + p['bv'])[src].reshape(-1, heads, out)
    e = (ea @ p['We']).reshape(-1, heads, out)
    k = k + e
    v = v + e
    alpha = (q * k).sum(-1) / np.sqrt(out)
    m = jax.ops.segment_max(alpha, dst, num_segments=Nn)
    ex = jnp.exp(alpha - m[dst])
    den = jax.ops.segment_sum(ex, dst, num_segments=Nn)
    a = ex / (den[dst] + 1e-16)
    agg = jax.ops.segment_sum(v * a[..., None], dst, num_segments=Nn).reshape(Nn, heads * out)
    return agg + x @ p['Ws'] + p['bs']


def _bn(x, g, b):
    mu = x.mean(0)
    var = x.var(0)
    return g * (x - mu) / jnp.sqrt(var + 1e-5) + b


def _head_kernel(h_ref, w1_ref, b1_ref, w2_ref, b2_ref, pa_ref, o_ref):
    h = h_ref[...]
    h = jnp.dot(h, w1_ref[...], preferred_element_type=jnp.float32) + b1_ref[...]
    h = jnp.where(h > 0, h, pa_ref[0] * h)
    h = jnp.dot(h, w2_ref[...], preferred_element_type=jnp.float32) + b2_ref[...]
    mx = jnp.max(h, axis=-1, keepdims=True)
    lse = mx + jnp.log(jnp.sum(jnp.exp(h - mx), axis=-1, keepdims=True))
    o_ref[...] = h - lse


def kernel(x, edge_index, edge_attr, batch, params):
    p = params
    src, dst = edge_index[0], edge_index[1]
    x1 = _bn(_transformer_conv(x, src, dst, edge_attr, p['c1'], 4, 32), p['bn1g'], p['bn1b'])
    x2 = _bn(_transformer_conv(x1, src, dst, edge_attr, p['c2'], 1, 128), p['bn2g'], p['bn2b'])
    x3 = _bn(_transformer_conv(x2, src, dst, edge_attr, p['c3'], 1, 128), p['bn3g'], p['bn3b'])
    x_add = jax.ops.segment_sum(x3, batch, num_segments=G)
    x_max = jax.ops.segment_max(x3, batch, num_segments=G)
    cnt = jax.ops.segment_sum(jnp.ones((x3.shape[0],), jnp.float32), batch, num_segments=G)
    x_mean = x_add / jnp.maximum(cnt, 1.0)[:, None]
    h = jnp.concatenate([x_add, x_max, x_mean], axis=1)
    out = pl.pallas_call(
        _head_kernel,
        out_shape=jax.ShapeDtypeStruct((G, 2), jnp.float32),
    )(h, p['W1'], p['b1'].reshape(1, -1), p['W2'], p['b2'].reshape(1, -1),
      jnp.full((1,), p['pa'], jnp.float32))
    return out
